# baseline (speedup 1.0000x reference)
import functools

import jax
import jax.numpy as jnp
from jax import lax
from jax.experimental import pallas as pl
from jax.experimental.pallas import tpu as pltpu
from jax.experimental.pallas import tpu_sc as plsc

VOCAB = 1000000
DIM = 64
BATCH = 4096
SEQ = 200
NPAIR = VOCAB // 2
VT_FULL = VOCAB // 128
VT_MAIN = (VT_FULL // 32) * 32
NW = 32
R = 2

_mesh = plsc.VectorSubcoreMesh(core_axis_name="c", subcore_axis_name="s")
_params = pltpu.CompilerParams(use_tc_tiling_on_sc=True,
                               needs_layout_passes=False)


@functools.partial(
    pl.kernel,
    out_type=jax.ShapeDtypeStruct((NPAIR, 128), jnp.float32),
    mesh=_mesh,
    scratch_types=[
        [pltpu.VMEM((64, 128), jnp.float32)] * R,
        [pltpu.VMEM((64, 128), jnp.float32)] * R,
        [pltpu.SemaphoreType.DMA] * R,
        [pltpu.SemaphoreType.DMA] * R,
    ],
    compiler_params=_params,
)
def _convert(tableT, tail_packed, t2, pblk, tblk, rsem, wsem):
    wid = lax.axis_index("s") * 2 + lax.axis_index("c")
    iota = lax.iota(jnp.int32, 16)

    def read_tile(t, b):
        pltpu.async_copy(
            tableT.at[pl.ds(0, 64), pl.ds(128 * t, 128)], pblk[b], rsem[b])

    def transpose_block(src, dst):
        def vblock(v0):
            for d0 in range(0, 64, 16):
                dv = iota + d0

                def kbody(k, dv=dv, v0=v0):
                    m = lax.bitwise_and(iota + k, 15)
                    vloc = v0 + m
                    val = plsc.load_gather(src, [dv, vloc])
                    qv = lax.shift_right_logical(vloc, 1)
                    cv = dv + lax.shift_left(lax.bitwise_and(vloc, 1), 6)
                    plsc.store_scatter(dst, [qv, cv], val)

                plsc.parallel_loop(0, 16, unroll=8)(kbody)
        plsc.parallel_loop(0, 128, step=16)(vblock)

    n_main = VT_MAIN // NW

    for i in range(R - 1):
        read_tile(wid + NW * i, i)

    def step(i, b):
        t = wid + NW * i

        @pl.when(i + (R - 1) < n_main)
        def _():
            read_tile(wid + NW * (i + R - 1), (b + R - 1) % R)

        pltpu.make_async_copy(
            tableT.at[pl.ds(0, 64), pl.ds(0, 128)], pblk[b], rsem[b]).wait()

        @pl.when(i >= R)
        def _():
            pltpu.make_async_copy(
                tblk[b], t2.at[pl.ds(0, 64)], wsem[b]).wait()

        transpose_block(pblk[b], tblk[b])
        pltpu.async_copy(tblk[b], t2.at[pl.ds(64 * t, 64)], wsem[b])

    def group(i0):
        for r in range(R):
            step(i0 + r, r)

    pl.loop(0, n_main, step=R)(group)

    for b in range(R):
        pltpu.make_async_copy(tblk[b], t2.at[pl.ds(0, 64)], wsem[b]).wait()

    @pl.when(wid < VT_FULL - VT_MAIN)
    def _():
        t = VT_MAIN + wid
        pltpu.sync_copy(tableT.at[pl.ds(0, 64), pl.ds(128 * t, 128)],
                        pblk[0])
        transpose_block(pblk[0], tblk[0])
        pltpu.sync_copy(tblk[0], t2.at[pl.ds(64 * t, 64)])

    @pl.when(wid == 4)
    def _():
        pltpu.sync_copy(tail_packed, tblk[0].at[pl.ds(0, 32)])
        pltpu.sync_copy(tblk[0].at[pl.ds(0, 32)],
                        t2.at[pl.ds(64 * VT_FULL, 32)])


@functools.partial(
    pl.kernel,
    out_type=jax.ShapeDtypeStruct((SEQ, DIM, BATCH), jnp.float32),
    mesh=_mesh,
    scratch_types=[
        pltpu.VMEM((SEQ, 128), jnp.int32),
        [pltpu.VMEM((128,), jnp.int32)] * R,
        [pltpu.VMEM((128,), jnp.int32)] * R,
        [pltpu.VMEM((128, 128), jnp.float32)] * R,
        [pltpu.VMEM((1, DIM, 128), jnp.float32)] * R,
        [pltpu.SemaphoreType.DMA] * R,
        [pltpu.SemaphoreType.DMA] * R,
    ],
    compiler_params=_params,
)
def _gather_t(idsT, t2, out, ids_slab, idx2, par64, g, ostage, gsem, wsem):
    wid = lax.axis_index("s") * 2 + lax.axis_index("c")
    iota = lax.iota(jnp.int32, 16)

    pltpu.sync_copy(idsT.at[pl.ds(0, SEQ), pl.ds(128 * wid, 128)], ids_slab)

    def prep(s, b):
        for c0 in range(0, 128, 16):
            v = ids_slab[s, pl.ds(c0, 16)]
            idx2[b][pl.ds(c0, 16)] = lax.shift_right_logical(v, 1)
            par64[b][pl.ds(c0, 16)] = lax.shift_left(
                lax.bitwise_and(v, 1), 6)
        pltpu.async_copy(t2.at[idx2[b]], g[b], gsem[b])

    zero16 = lax.iota(jnp.int32, 16) * 0

    def transpose_s(b):
        def dblock(d0):
            for b0 in range(0, 128, 16):
                par_v = par64[b][pl.ds(b0, 16)]
                row_v = iota + b0
                pd = par_v + d0

                def kbody(k, pd=pd, row_v=row_v, d0=d0):
                    m = lax.bitwise_and(iota + k, 15)
                    val = plsc.load_gather(g[b], [row_v, pd + m])
                    plsc.store_scatter(ostage[b], [zero16, d0 + m, row_v],
                                       val)

                plsc.parallel_loop(0, 16, unroll=8)(kbody)
        plsc.parallel_loop(0, DIM, step=16)(dblock)

    for s in range(R - 1):
        prep(s, s)

    def step(s, b):
        @pl.when(s + (R - 1) < SEQ)
        def _():
            prep(s + R - 1, (b + R - 1) % R)

        pltpu.make_async_copy(t2.at[idx2[b]], g[b], gsem[b]).wait()

        @pl.when(s >= R)
        def _():
            pltpu.make_async_copy(
                ostage[b],
                out.at[pl.ds(0, 1), pl.ds(0, DIM), pl.ds(0, 128)],
                wsem[b]).wait()

        transpose_s(b)
        pltpu.async_copy(ostage[b],
                         out.at[pl.ds(s, 1), pl.ds(0, DIM),
                                pl.ds(128 * wid, 128)],
                         wsem[b])

    def group(s0):
        for r in range(R):
            step(s0 + r, r)

    pl.loop(0, SEQ, step=R)(group)

    for b in range(R):
        pltpu.make_async_copy(
            ostage[b], out.at[pl.ds(0, 1), pl.ds(0, DIM), pl.ds(0, 128)],
            wsem[b]).wait()


def kernel(token_ids, weights):
    tail_packed = weights[VT_FULL * 128:, :].reshape(32, 128)
    t2 = _convert(weights.T, tail_packed)
    out_t = _gather_t(token_ids.T, t2)
    return jnp.transpose(out_t, (2, 0, 1))


# speedup vs baseline: 1.1236x; 1.1236x over previous
import functools

import jax
import jax.numpy as jnp
from jax import lax
from jax.experimental import pallas as pl
from jax.experimental.pallas import tpu as pltpu
from jax.experimental.pallas import tpu_sc as plsc

VOCAB = 1000000
DIM = 64
BATCH = 4096
SEQ = 200
NPAIR = VOCAB // 2
VT_FULL = VOCAB // 128
VT_MAIN = (VT_FULL // 32) * 32
NW = 32
R = 2

_mesh = plsc.VectorSubcoreMesh(core_axis_name="c", subcore_axis_name="s")
_params = pltpu.CompilerParams(use_tc_tiling_on_sc=True,
                               needs_layout_passes=False)


@functools.partial(
    pl.kernel,
    out_type=jax.ShapeDtypeStruct((NPAIR, 128), jnp.float32),
    mesh=_mesh,
    scratch_types=[
        [pltpu.VMEM((64, 128), jnp.float32)] * R,
        [pltpu.VMEM((64, 128), jnp.float32)] * R,
        [pltpu.SemaphoreType.DMA] * R,
        [pltpu.SemaphoreType.DMA] * R,
    ],
    compiler_params=_params,
)
def _convert(tableT, tail_packed, t2, pblk, tblk, rsem, wsem):
    wid = lax.axis_index("s") * 2 + lax.axis_index("c")
    iota = lax.iota(jnp.int32, 16)

    def read_tile(t, b):
        pltpu.async_copy(
            tableT.at[pl.ds(0, 64), pl.ds(128 * t, 128)], pblk[b], rsem[b])

    def transpose_block(src, dst):
        def vblock(v0):
            colv = v0 + iota
            qv = lax.shift_right_logical(colv, 1)
            h = lax.shift_left(lax.bitwise_and(colv, 1), 6)
            for d0 in range(0, 64, 16):
                hd0 = h + d0

                def kbody(k, hd0=hd0, d0=d0):
                    m = lax.bitwise_and(iota + k, 15)
                    val = plsc.load_gather(src, [d0 + m, colv])
                    plsc.store_scatter(dst, [qv, hd0 + m], val)

                plsc.parallel_loop(0, 16, unroll=4)(kbody)
        plsc.parallel_loop(0, 128, step=16)(vblock)

    n_main = VT_MAIN // NW

    for i in range(R - 1):
        read_tile(wid + NW * i, i)

    def step(i, b):
        t = wid + NW * i

        @pl.when(i + (R - 1) < n_main)
        def _():
            read_tile(wid + NW * (i + R - 1), (b + R - 1) % R)

        pltpu.make_async_copy(
            tableT.at[pl.ds(0, 64), pl.ds(0, 128)], pblk[b], rsem[b]).wait()

        @pl.when(i >= R)
        def _():
            pltpu.make_async_copy(
                tblk[b], t2.at[pl.ds(0, 64)], wsem[b]).wait()

        transpose_block(pblk[b], tblk[b])
        pltpu.async_copy(tblk[b], t2.at[pl.ds(64 * t, 64)], wsem[b])

    def group(i0):
        for r in range(R):
            step(i0 + r, r)

    pl.loop(0, n_main, step=R)(group)

    for b in range(R):
        pltpu.make_async_copy(tblk[b], t2.at[pl.ds(0, 64)], wsem[b]).wait()

    @pl.when(wid < VT_FULL - VT_MAIN)
    def _():
        t = VT_MAIN + wid
        pltpu.sync_copy(tableT.at[pl.ds(0, 64), pl.ds(128 * t, 128)],
                        pblk[0])
        transpose_block(pblk[0], tblk[0])
        pltpu.sync_copy(tblk[0], t2.at[pl.ds(64 * t, 64)])

    @pl.when(wid == 4)
    def _():
        pltpu.sync_copy(tail_packed, tblk[0].at[pl.ds(0, 32)])
        pltpu.sync_copy(tblk[0].at[pl.ds(0, 32)],
                        t2.at[pl.ds(64 * VT_FULL, 32)])


@functools.partial(
    pl.kernel,
    out_type=jax.ShapeDtypeStruct((SEQ, DIM, BATCH), jnp.float32),
    mesh=_mesh,
    scratch_types=[
        pltpu.VMEM((SEQ, 128), jnp.int32),
        [pltpu.VMEM((128,), jnp.int32)] * R,
        [pltpu.VMEM((128,), jnp.int32)] * R,
        [pltpu.VMEM((128, 128), jnp.float32)] * R,
        [pltpu.VMEM((1, DIM, 128), jnp.float32)] * R,
        [pltpu.SemaphoreType.DMA] * R,
        [pltpu.SemaphoreType.DMA] * R,
    ],
    compiler_params=_params,
)
def _gather_t(idsT, t2, out, ids_slab, idx2, par64, g, ostage, gsem, wsem):
    wid = lax.axis_index("s") * 2 + lax.axis_index("c")
    iota = lax.iota(jnp.int32, 16)

    pltpu.sync_copy(idsT.at[pl.ds(0, SEQ), pl.ds(128 * wid, 128)], ids_slab)

    def prep(s, b):
        for c0 in range(0, 128, 16):
            v = ids_slab[s, pl.ds(c0, 16)]
            idx2[b][pl.ds(c0, 16)] = lax.shift_right_logical(v, 1)
            par64[b][pl.ds(c0, 16)] = lax.shift_left(
                lax.bitwise_and(v, 1), 6)
        pltpu.async_copy(t2.at[idx2[b]], g[b], gsem[b])

    zero16 = lax.iota(jnp.int32, 16) * 0

    def transpose_s(b):
        def dblock(d0):
            for b0 in range(0, 128, 16):
                par_v = par64[b][pl.ds(b0, 16)]
                row_v = iota + b0
                pd = par_v + d0

                def kbody(k, pd=pd, row_v=row_v, d0=d0):
                    m = lax.bitwise_and(iota + k, 15)
                    val = plsc.load_gather(g[b], [row_v, pd + m])
                    plsc.store_scatter(ostage[b], [zero16, d0 + m, row_v],
                                       val)

                plsc.parallel_loop(0, 16, unroll=4)(kbody)
        plsc.parallel_loop(0, DIM, step=16)(dblock)

    for s in range(R - 1):
        prep(s, s)

    def step(s, b):
        @pl.when(s + (R - 1) < SEQ)
        def _():
            prep(s + R - 1, (b + R - 1) % R)

        pltpu.make_async_copy(t2.at[idx2[b]], g[b], gsem[b]).wait()

        @pl.when(s >= R)
        def _():
            pltpu.make_async_copy(
                ostage[b],
                out.at[pl.ds(0, 1), pl.ds(0, DIM), pl.ds(0, 128)],
                wsem[b]).wait()

        transpose_s(b)
        pltpu.async_copy(ostage[b],
                         out.at[pl.ds(s, 1), pl.ds(0, DIM),
                                pl.ds(128 * wid, 128)],
                         wsem[b])

    def group(s0):
        for r in range(R):
            step(s0 + r, r)

    pl.loop(0, SEQ, step=R)(group)

    for b in range(R):
        pltpu.make_async_copy(
            ostage[b], out.at[pl.ds(0, 1), pl.ds(0, DIM), pl.ds(0, 128)],
            wsem[b]).wait()


def kernel(token_ids, weights):
    tail_packed = weights[VT_FULL * 128:, :].reshape(32, 128)
    t2 = _convert(weights.T, tail_packed)
    out_t = _gather_t(token_ids.T, t2)
    return jnp.transpose(out_t, (2, 0, 1))


# speedup vs baseline: 1.2096x; 1.0766x over previous
import functools

import jax
import jax.numpy as jnp
from jax import lax
from jax.experimental import pallas as pl
from jax.experimental.pallas import tpu as pltpu
from jax.experimental.pallas import tpu_sc as plsc

VOCAB = 1000000
DIM = 64
BATCH = 4096
SEQ = 200
NPAIR = VOCAB // 2
VT_FULL = VOCAB // 128
VT_MAIN = (VT_FULL // 32) * 32
NW = 32
R = 2
R2 = 4

_mesh = plsc.VectorSubcoreMesh(core_axis_name="c", subcore_axis_name="s")
_params = pltpu.CompilerParams(use_tc_tiling_on_sc=True,
                               needs_layout_passes=False)


@functools.partial(
    pl.kernel,
    out_type=jax.ShapeDtypeStruct((NPAIR, 128), jnp.float32),
    mesh=_mesh,
    scratch_types=[
        [pltpu.VMEM((64, 128), jnp.float32)] * R,
        [pltpu.VMEM((64, 128), jnp.float32)] * R,
        [pltpu.SemaphoreType.DMA] * R,
        [pltpu.SemaphoreType.DMA] * R,
    ],
    compiler_params=_params,
)
def _convert(tableT, tail_packed, t2, pblk, tblk, rsem, wsem):
    wid = lax.axis_index("s") * 2 + lax.axis_index("c")
    iota = lax.iota(jnp.int32, 16)

    def read_tile(t, b):
        pltpu.async_copy(
            tableT.at[pl.ds(0, 64), pl.ds(128 * t, 128)], pblk[b], rsem[b])

    def transpose_block(src, dst):
        def vblock(v0):
            colv = v0 + iota
            qv = lax.shift_right_logical(colv, 1)
            h = lax.shift_left(lax.bitwise_and(colv, 1), 6)
            for d0 in range(0, 64, 16):
                hd0 = h + d0

                def kbody(k, hd0=hd0, d0=d0):
                    m = lax.bitwise_and(iota + k, 15)
                    val = plsc.load_gather(src, [d0 + m, colv])
                    plsc.store_scatter(dst, [qv, hd0 + m], val)

                plsc.parallel_loop(0, 16, unroll=4)(kbody)
        plsc.parallel_loop(0, 128, step=16)(vblock)

    n_main = VT_MAIN // NW

    for i in range(R - 1):
        read_tile(wid + NW * i, i)

    def step(i, b):
        t = wid + NW * i

        @pl.when(i + (R - 1) < n_main)
        def _():
            read_tile(wid + NW * (i + R - 1), (b + R - 1) % R)

        pltpu.make_async_copy(
            tableT.at[pl.ds(0, 64), pl.ds(0, 128)], pblk[b], rsem[b]).wait()

        @pl.when(i >= R)
        def _():
            pltpu.make_async_copy(
                tblk[b], t2.at[pl.ds(0, 64)], wsem[b]).wait()

        transpose_block(pblk[b], tblk[b])
        pltpu.async_copy(tblk[b], t2.at[pl.ds(64 * t, 64)], wsem[b])

    def group(i0):
        for r in range(R):
            step(i0 + r, r)

    pl.loop(0, n_main, step=R)(group)

    for b in range(R):
        pltpu.make_async_copy(tblk[b], t2.at[pl.ds(0, 64)], wsem[b]).wait()

    @pl.when(wid < VT_FULL - VT_MAIN)
    def _():
        t = VT_MAIN + wid
        pltpu.sync_copy(tableT.at[pl.ds(0, 64), pl.ds(128 * t, 128)],
                        pblk[0])
        transpose_block(pblk[0], tblk[0])
        pltpu.sync_copy(tblk[0], t2.at[pl.ds(64 * t, 64)])

    @pl.when(wid == 4)
    def _():
        pltpu.sync_copy(tail_packed, tblk[0].at[pl.ds(0, 32)])
        pltpu.sync_copy(tblk[0].at[pl.ds(0, 32)],
                        t2.at[pl.ds(64 * VT_FULL, 32)])


@functools.partial(
    pl.kernel,
    out_type=jax.ShapeDtypeStruct((SEQ, DIM, BATCH), jnp.float32),
    mesh=_mesh,
    scratch_types=[
        pltpu.VMEM((SEQ, 128), jnp.int32),
        [pltpu.VMEM((128,), jnp.int32)] * R2,
        [pltpu.VMEM((128,), jnp.int32)] * R2,
        [pltpu.VMEM((128, 128), jnp.float32)] * R2,
        [pltpu.VMEM((1, DIM, 128), jnp.float32)] * R2,
        [pltpu.SemaphoreType.DMA] * R2,
        [pltpu.SemaphoreType.DMA] * R2,
    ],
    compiler_params=_params,
)
def _gather_t(idsT, t2, out, ids_slab, idx2, par64, g, ostage, gsem, wsem):
    wid = lax.axis_index("s") * 2 + lax.axis_index("c")
    iota = lax.iota(jnp.int32, 16)

    pltpu.sync_copy(idsT.at[pl.ds(0, SEQ), pl.ds(128 * wid, 128)], ids_slab)

    def prep(s, b):
        for c0 in range(0, 128, 16):
            v = ids_slab[s, pl.ds(c0, 16)]
            idx2[b][pl.ds(c0, 16)] = lax.shift_right_logical(v, 1)
            par64[b][pl.ds(c0, 16)] = lax.shift_left(
                lax.bitwise_and(v, 1), 6)
        pltpu.async_copy(t2.at[idx2[b]], g[b], gsem[b])

    zero16 = lax.iota(jnp.int32, 16) * 0

    def transpose_s(b):
        def dblock(d0):
            for b0 in range(0, 128, 16):
                par_v = par64[b][pl.ds(b0, 16)]
                row_v = iota + b0
                pd = par_v + d0

                def kbody(k, pd=pd, row_v=row_v, d0=d0):
                    m = lax.bitwise_and(iota + k, 15)
                    val = plsc.load_gather(g[b], [row_v, pd + m])
                    plsc.store_scatter(ostage[b], [zero16, d0 + m, row_v],
                                       val)

                plsc.parallel_loop(0, 16, unroll=4)(kbody)
        plsc.parallel_loop(0, DIM, step=16)(dblock)

    for s in range(R2 - 1):
        prep(s, s)

    def step(s, b):
        @pl.when(s + (R2 - 1) < SEQ)
        def _():
            prep(s + R2 - 1, (b + R2 - 1) % R2)

        pltpu.make_async_copy(t2.at[idx2[b]], g[b], gsem[b]).wait()

        @pl.when(s >= R2)
        def _():
            pltpu.make_async_copy(
                ostage[b],
                out.at[pl.ds(0, 1), pl.ds(0, DIM), pl.ds(0, 128)],
                wsem[b]).wait()

        transpose_s(b)
        pltpu.async_copy(ostage[b],
                         out.at[pl.ds(s, 1), pl.ds(0, DIM),
                                pl.ds(128 * wid, 128)],
                         wsem[b])

    def group(s0):
        for r in range(R2):
            step(s0 + r, r)

    pl.loop(0, SEQ, step=R2)(group)

    for b in range(R2):
        pltpu.make_async_copy(
            ostage[b], out.at[pl.ds(0, 1), pl.ds(0, DIM), pl.ds(0, 128)],
            wsem[b]).wait()


def kernel(token_ids, weights):
    tail_packed = weights[VT_FULL * 128:, :].reshape(32, 128)
    t2 = _convert(weights.T, tail_packed)
    out_t = _gather_t(token_ids.T, t2)
    return jnp.transpose(out_t, (2, 0, 1))


# speedup vs baseline: 1.3934x; 1.1519x over previous
import functools

import jax
import jax.numpy as jnp
from jax import lax
from jax.experimental import pallas as pl
from jax.experimental.pallas import tpu as pltpu
from jax.experimental.pallas import tpu_sc as plsc

VOCAB = 1000000
DIM = 64
BATCH = 4096
SEQ = 200
NPAIR = VOCAB // 2
VT_FULL = VOCAB // 128
VT_MAIN = (VT_FULL // 32) * 32
NW = 32
R = 4
R2 = 4

_mesh = plsc.VectorSubcoreMesh(core_axis_name="c", subcore_axis_name="s")
_params = pltpu.CompilerParams(use_tc_tiling_on_sc=True,
                               needs_layout_passes=False)


@functools.partial(
    pl.kernel,
    out_type=jax.ShapeDtypeStruct((NPAIR, 128), jnp.float32),
    mesh=_mesh,
    scratch_types=[
        [pltpu.VMEM((64, 128), jnp.float32)] * R,
        [pltpu.VMEM((64, 128), jnp.float32)] * R,
        [pltpu.SemaphoreType.DMA] * R,
        [pltpu.SemaphoreType.DMA] * R,
    ],
    compiler_params=_params,
)
def _convert(tableT, tail_packed, t2, pblk, tblk, rsem, wsem):
    wid = lax.axis_index("s") * 2 + lax.axis_index("c")
    iota = lax.iota(jnp.int32, 16)

    def read_tile(t, b):
        pltpu.async_copy(
            tableT.at[pl.ds(0, 64), pl.ds(128 * t, 128)], pblk[b], rsem[b])

    def transpose_block(src, dst):
        def vblock(v0):
            colv = v0 + iota
            qv = lax.shift_right_logical(colv, 1)
            h = lax.shift_left(lax.bitwise_and(colv, 1), 6)
            for d0 in range(0, 64, 16):
                hd0 = h + d0

                def kbody(k, hd0=hd0, d0=d0):
                    m = lax.bitwise_and(iota + k, 15)
                    val = plsc.load_gather(src, [d0 + m, colv])
                    plsc.store_scatter(dst, [qv, hd0 + m], val)

                plsc.parallel_loop(0, 16, unroll=4)(kbody)
        plsc.parallel_loop(0, 128, step=16)(vblock)

    n_main = VT_MAIN // NW

    for i in range(R - 1):
        read_tile(wid + NW * i, i)

    def step(i, b):
        t = wid + NW * i

        @pl.when(i + (R - 1) < n_main)
        def _():
            read_tile(wid + NW * (i + R - 1), (b + R - 1) % R)

        pltpu.make_async_copy(
            tableT.at[pl.ds(0, 64), pl.ds(0, 128)], pblk[b], rsem[b]).wait()

        @pl.when(i >= R)
        def _():
            pltpu.make_async_copy(
                tblk[b], t2.at[pl.ds(0, 64)], wsem[b]).wait()

        transpose_block(pblk[b], tblk[b])
        pltpu.async_copy(tblk[b], t2.at[pl.ds(64 * t, 64)], wsem[b])

    def group(i0):
        for r in range(R):
            step(i0 + r, r)

    pl.loop(0, n_main, step=R)(group)

    for b in range(R):
        pltpu.make_async_copy(tblk[b], t2.at[pl.ds(0, 64)], wsem[b]).wait()

    @pl.when(wid < VT_FULL - VT_MAIN)
    def _():
        t = VT_MAIN + wid
        pltpu.sync_copy(tableT.at[pl.ds(0, 64), pl.ds(128 * t, 128)],
                        pblk[0])
        transpose_block(pblk[0], tblk[0])
        pltpu.sync_copy(tblk[0], t2.at[pl.ds(64 * t, 64)])

    @pl.when(wid == 4)
    def _():
        pltpu.sync_copy(tail_packed, tblk[0].at[pl.ds(0, 32)])
        pltpu.sync_copy(tblk[0].at[pl.ds(0, 32)],
                        t2.at[pl.ds(64 * VT_FULL, 32)])


@functools.partial(
    pl.kernel,
    out_type=jax.ShapeDtypeStruct((SEQ, DIM, BATCH), jnp.float32),
    mesh=_mesh,
    scratch_types=[
        pltpu.VMEM((SEQ, 128), jnp.int32),
        [pltpu.VMEM((128,), jnp.int32)] * R2,
        [pltpu.VMEM((128,), jnp.int32)] * R2,
        [pltpu.VMEM((128, 128), jnp.float32)] * R2,
        [pltpu.VMEM((1, DIM, 128), jnp.float32)] * R2,
        [pltpu.SemaphoreType.DMA] * R2,
        [pltpu.SemaphoreType.DMA] * R2,
    ],
    compiler_params=_params,
)
def _gather_t(idsT, t2, out, ids_slab, idx2, par64, g, ostage, gsem, wsem):
    wid = lax.axis_index("s") * 2 + lax.axis_index("c")
    iota = lax.iota(jnp.int32, 16)

    pltpu.sync_copy(idsT.at[pl.ds(0, SEQ), pl.ds(128 * wid, 128)], ids_slab)

    def prep(s, b):
        for c0 in range(0, 128, 16):
            v = ids_slab[s, pl.ds(c0, 16)]
            idx2[b][pl.ds(c0, 16)] = lax.shift_right_logical(v, 1)
            par64[b][pl.ds(c0, 16)] = lax.shift_left(
                lax.bitwise_and(v, 1), 6)
        pltpu.async_copy(t2.at[idx2[b]], g[b], gsem[b])

    zero16 = lax.iota(jnp.int32, 16) * 0

    def transpose_s(b):
        def dblock(d0):
            for b0 in range(0, 128, 16):
                par_v = par64[b][pl.ds(b0, 16)]
                row_v = iota + b0
                pd = par_v + d0

                def kbody(k, pd=pd, row_v=row_v, d0=d0):
                    m = lax.bitwise_and(iota + k, 15)
                    val = plsc.load_gather(g[b], [row_v, pd + m])
                    plsc.store_scatter(ostage[b], [zero16, d0 + m, row_v],
                                       val)

                plsc.parallel_loop(0, 16, unroll=4)(kbody)
        plsc.parallel_loop(0, DIM, step=16)(dblock)

    for s in range(R2 - 1):
        prep(s, s)

    def step(s, b):
        @pl.when(s + (R2 - 1) < SEQ)
        def _():
            prep(s + R2 - 1, (b + R2 - 1) % R2)

        pltpu.make_async_copy(t2.at[idx2[b]], g[b], gsem[b]).wait()

        @pl.when(s >= R2)
        def _():
            pltpu.make_async_copy(
                ostage[b],
                out.at[pl.ds(0, 1), pl.ds(0, DIM), pl.ds(0, 128)],
                wsem[b]).wait()

        transpose_s(b)
        pltpu.async_copy(ostage[b],
                         out.at[pl.ds(s, 1), pl.ds(0, DIM),
                                pl.ds(128 * wid, 128)],
                         wsem[b])

    def group(s0):
        for r in range(R2):
            step(s0 + r, r)

    pl.loop(0, SEQ, step=R2)(group)

    for b in range(R2):
        pltpu.make_async_copy(
            ostage[b], out.at[pl.ds(0, 1), pl.ds(0, DIM), pl.ds(0, 128)],
            wsem[b]).wait()


def kernel(token_ids, weights):
    tail_packed = weights[VT_FULL * 128:, :].reshape(32, 128)
    t2 = _convert(weights.T, tail_packed)
    out_t = _gather_t(token_ids.T, t2)
    return jnp.transpose(out_t, (2, 0, 1))
